# Initial kernel scaffold; baseline (speedup 1.0000x reference)
import jax
import jax.numpy as jnp
from jax.experimental import pallas as pl


def kernel(x, edge_index, Wn1, Ws1, Wimp1, bimp1, b1, Wn2, Ws2, Wimp2, bimp2, b2, Wm1, bm1, Wm2, bm2, Wm3, bm3):
    raise NotImplementedError("write your pallas kernel here")


# speedup vs baseline: 5.5315x; 5.5315x over previous
import functools

import jax
import jax.numpy as jnp
from jax import lax
from jax.experimental import pallas as pl
from jax.experimental.pallas import tpu as pltpu
from jax.experimental.pallas import tpu_sc as plsc

_NC = 2
_NS = 16
_NW = _NC * _NS
_CH = 128

_B = 1000


def _sigmoid(v):
    return 1.0 / (1.0 + jnp.exp(-v))


def _pre_body(x_ref, wn_ref, ws_ref, wimp_ref, bimp_ref, msg_ref, xs_ref):
    x = x_ref[...]
    xn = jnp.dot(x, wn_ref[...], preferred_element_type=jnp.float32)
    logit = jnp.dot(xn, wimp_ref[...], preferred_element_type=jnp.float32) + bimp_ref[...]
    msg_ref[...] = _sigmoid(logit) * jnp.maximum(xn, 0.0)
    xs_ref[...] = jnp.dot(x, ws_ref[...], preferred_element_type=jnp.float32)


def _mid_body(parts_ref, xs_ref, b_ref, wn_ref, ws_ref, wimp_ref, bimp_ref,
              msg_ref, xs2_ref):
    p = parts_ref[...]
    h = jnp.maximum(p[0] + p[1] + xs_ref[...] + b_ref[...], 0.0)
    xn = jnp.dot(h, wn_ref[...], preferred_element_type=jnp.float32)
    logit = jnp.dot(xn, wimp_ref[...], preferred_element_type=jnp.float32) + bimp_ref[...]
    msg_ref[...] = _sigmoid(logit) * jnp.maximum(xn, 0.0)
    xs2_ref[...] = jnp.dot(h, ws_ref[...], preferred_element_type=jnp.float32)


def _post_body(parts_ref, xs_ref, b_ref, wm1_ref, bm1_ref, wm2_ref, bm2_ref,
               wm3_ref, bm3_ref, out_ref):
    p = parts_ref[...]
    h = jnp.maximum(p[0] + p[1] + xs_ref[...] + b_ref[...], 0.0)
    m = jnp.maximum(jnp.dot(h, wm1_ref[...], preferred_element_type=jnp.float32)
                    + bm1_ref[...], 0.0)
    m = jnp.maximum(jnp.dot(m, wm2_ref[...], preferred_element_type=jnp.float32)
                    + bm2_ref[...], 0.0)
    out_ref[...] = _sigmoid(
        jnp.dot(m, wm3_ref[...], preferred_element_type=jnp.float32)
        + bm3_ref[...])


def _full(shape):
    return pl.BlockSpec(shape, lambda i: (0,) * len(shape))


def _rows(shape):
    return pl.BlockSpec(shape, lambda i: (i,) + (0,) * (len(shape) - 1))


def _parts_spec(h):
    return pl.BlockSpec((_NC, _B, h), lambda i: (0, i, 0))


def _pre_call(x, Wn, Ws, Wimp, bimp):
    n, d = x.shape
    h = Wn.shape[1]
    grid = n // _B
    return pl.pallas_call(
        _pre_body,
        grid=(grid,),
        in_specs=[_rows((_B, d)), _full((d, h)), _full((d, h)),
                  _full((d, 1)), _full((1, 1))],
        out_specs=[_rows((_B, h)), _rows((_B, h))],
        out_shape=[jax.ShapeDtypeStruct((n, h), jnp.float32),
                   jax.ShapeDtypeStruct((n, h), jnp.float32)],
    )(x, Wn, Ws, Wimp.reshape(d, 1), bimp.reshape(1, 1))


def _mid_call(parts, xs, b, Wn, Ws, Wimp, bimp):
    n, h = xs.shape
    h2 = Wn.shape[1]
    grid = n // _B
    return pl.pallas_call(
        _mid_body,
        grid=(grid,),
        in_specs=[_parts_spec(h), _rows((_B, h)), _full((1, h)),
                  _full((h, h2)), _full((h, h2)), _full((h2, 1)), _full((1, 1))],
        out_specs=[_rows((_B, h2)), _rows((_B, h2))],
        out_shape=[jax.ShapeDtypeStruct((n, h2), jnp.float32),
                   jax.ShapeDtypeStruct((n, h2), jnp.float32)],
    )(parts, xs, b.reshape(1, h), Wn, Ws, Wimp.reshape(h, 1),
      bimp.reshape(1, 1))


def _post_call(parts, xs, b, Wm1, bm1, Wm2, bm2, Wm3, bm3):
    n, h = xs.shape
    d1, d2, d3 = Wm1.shape[1], Wm2.shape[1], Wm3.shape[1]
    grid = n // _B
    return pl.pallas_call(
        _post_body,
        grid=(grid,),
        in_specs=[_parts_spec(h), _rows((_B, h)), _full((1, h)),
                  _full((h, d1)), _full((1, d1)),
                  _full((d1, d2)), _full((1, d2)),
                  _full((d2, d3)), _full((1, d3))],
        out_specs=_rows((_B, d3)),
        out_shape=jax.ShapeDtypeStruct((n, d3), jnp.float32),
    )(parts, xs, b.reshape(1, h), Wm1, bm1.reshape(1, d1),
      Wm2, bm2.reshape(1, d2), Wm3, bm3.reshape(1, d3))


@functools.lru_cache(maxsize=None)
def _make_agg(nchunk, acc_rows, h):
    rows_per_tile = acc_rows // _NS
    mesh = plsc.VectorSubcoreMesh(core_axis_name="c", subcore_axis_name="s")

    @functools.partial(
        pl.kernel,
        mesh=mesh,
        out_type=jax.ShapeDtypeStruct((_NC, acc_rows, h), jnp.float32),
        scratch_types=[
            pltpu.VMEM((nchunk, _CH), jnp.int32),
            pltpu.VMEM((nchunk, _CH), jnp.int32),
            pltpu.VMEM((_CH, h), jnp.float32),
            pltpu.VMEM_SHARED((acc_rows, h), jnp.float32),
            pltpu.SemaphoreType.DMA,
        ],
    )
    def agg(msg_hbm, srcb_hbm, dstb_hbm, zeros_hbm, out_hbm,
            src_v, dst_v, rows_v, acc, sem):
        c = lax.axis_index("c")
        s = lax.axis_index("s")
        wid = c * _NS + s
        pltpu.sync_copy(srcb_hbm.at[wid], src_v)
        pltpu.sync_copy(dstb_hbm.at[wid], dst_v)
        pltpu.sync_copy(zeros_hbm, acc.at[pl.ds(s * rows_per_tile, rows_per_tile)])
        plsc.subcore_barrier()

        def body(j, carry):
            pltpu.async_copy(msg_hbm.at[src_v.at[j]], rows_v, sem).wait()
            pltpu.sync_copy(rows_v, acc.at[dst_v.at[j]], add=True)
            return carry

        lax.fori_loop(0, nchunk, body, 0)
        plsc.subcore_barrier()
        pltpu.sync_copy(acc.at[pl.ds(s * rows_per_tile, rows_per_tile)],
                        out_hbm.at[c, pl.ds(s * rows_per_tile, rows_per_tile)])

    return agg


def kernel(x, edge_index, Wn1, Ws1, Wimp1, bimp1, b1, Wn2, Ws2, Wimp2, bimp2,
           b2, Wm1, bm1, Wm2, bm2, Wm3, bm3):
    n, d = x.shape
    e = edge_index.shape[1]
    h = Wn1.shape[1]

    nchunk = -(-e // (_NW * _CH))
    ep = _NW * _CH * nchunk
    acc_rows = (_NS * 8) * (-(-(n + 1) // (_NS * 8)))

    src = edge_index[0].astype(jnp.int32)
    dst = edge_index[1].astype(jnp.int32)
    pad = ep - e
    srcb = jnp.concatenate([src, jnp.zeros((pad,), jnp.int32)]).reshape(
        _NW, nchunk, _CH)
    dstb = jnp.concatenate([dst, jnp.full((pad,), n, jnp.int32)]).reshape(
        _NW, nchunk, _CH)
    zeros = jnp.zeros((acc_rows // _NS, h), jnp.float32)

    agg_fn = _make_agg(nchunk, acc_rows, h)

    msg1, xs1 = _pre_call(x, Wn1, Ws1, Wimp1, bimp1)
    parts1 = agg_fn(msg1, srcb, dstb, zeros)
    msg2, xs2 = _mid_call(parts1, xs1, b1, Wn2, Ws2, Wimp2, bimp2)
    parts2 = agg_fn(msg2, srcb, dstb, zeros)
    return _post_call(parts2, xs2, b2, Wm1, bm1, Wm2, bm2, Wm3, bm3)


# speedup vs baseline: 6.0250x; 1.0892x over previous
import functools

import jax
import jax.numpy as jnp
from jax import lax
from jax.experimental import pallas as pl
from jax.experimental.pallas import tpu as pltpu
from jax.experimental.pallas import tpu_sc as plsc

_NC = 2
_NS = 16
_NW = _NC * _NS
_CH = 128

_B = 1000


def _sigmoid(v):
    return 1.0 / (1.0 + jnp.exp(-v))


def _pre_body(x_ref, wn_ref, ws_ref, wimp_ref, bimp_ref, msg_ref, xs_ref):
    x = x_ref[...]
    xn = jnp.dot(x, wn_ref[...], preferred_element_type=jnp.float32)
    logit = jnp.dot(xn, wimp_ref[...], preferred_element_type=jnp.float32) + bimp_ref[...]
    msg_ref[...] = _sigmoid(logit) * jnp.maximum(xn, 0.0)
    xs_ref[...] = jnp.dot(x, ws_ref[...], preferred_element_type=jnp.float32)


def _mid_body(parts_ref, xs_ref, b_ref, wn_ref, ws_ref, wimp_ref, bimp_ref,
              msg_ref, xs2_ref):
    p = parts_ref[...]
    h = jnp.maximum(p[0] + p[1] + xs_ref[...] + b_ref[...], 0.0)
    xn = jnp.dot(h, wn_ref[...], preferred_element_type=jnp.float32)
    logit = jnp.dot(xn, wimp_ref[...], preferred_element_type=jnp.float32) + bimp_ref[...]
    msg_ref[...] = _sigmoid(logit) * jnp.maximum(xn, 0.0)
    xs2_ref[...] = jnp.dot(h, ws_ref[...], preferred_element_type=jnp.float32)


def _post_body(parts_ref, xs_ref, b_ref, wm1_ref, bm1_ref, wm2_ref, bm2_ref,
               wm3_ref, bm3_ref, out_ref):
    p = parts_ref[...]
    h = jnp.maximum(p[0] + p[1] + xs_ref[...] + b_ref[...], 0.0)
    m = jnp.maximum(jnp.dot(h, wm1_ref[...], preferred_element_type=jnp.float32)
                    + bm1_ref[...], 0.0)
    m = jnp.maximum(jnp.dot(m, wm2_ref[...], preferred_element_type=jnp.float32)
                    + bm2_ref[...], 0.0)
    out_ref[...] = _sigmoid(
        jnp.dot(m, wm3_ref[...], preferred_element_type=jnp.float32)
        + bm3_ref[...])


def _full(shape):
    return pl.BlockSpec(shape, lambda i: (0,) * len(shape))


def _rows(shape):
    return pl.BlockSpec(shape, lambda i: (i,) + (0,) * (len(shape) - 1))


def _parts_spec(h):
    return pl.BlockSpec((_NC, _B, h), lambda i: (0, i, 0))


def _pre_call(x, Wn, Ws, Wimp, bimp):
    n, d = x.shape
    h = Wn.shape[1]
    grid = n // _B
    return pl.pallas_call(
        _pre_body,
        grid=(grid,),
        in_specs=[_rows((_B, d)), _full((d, h)), _full((d, h)),
                  _full((d, 1)), _full((1, 1))],
        out_specs=[_rows((_B, h)), _rows((_B, h))],
        out_shape=[jax.ShapeDtypeStruct((n, h), jnp.float32),
                   jax.ShapeDtypeStruct((n, h), jnp.float32)],
    )(x, Wn, Ws, Wimp.reshape(d, 1), bimp.reshape(1, 1))


def _mid_call(parts, xs, b, Wn, Ws, Wimp, bimp):
    n, h = xs.shape
    h2 = Wn.shape[1]
    grid = n // _B
    return pl.pallas_call(
        _mid_body,
        grid=(grid,),
        in_specs=[_parts_spec(h), _rows((_B, h)), _full((1, h)),
                  _full((h, h2)), _full((h, h2)), _full((h2, 1)), _full((1, 1))],
        out_specs=[_rows((_B, h2)), _rows((_B, h2))],
        out_shape=[jax.ShapeDtypeStruct((n, h2), jnp.float32),
                   jax.ShapeDtypeStruct((n, h2), jnp.float32)],
    )(parts, xs, b.reshape(1, h), Wn, Ws, Wimp.reshape(h, 1),
      bimp.reshape(1, 1))


def _post_call(parts, xs, b, Wm1, bm1, Wm2, bm2, Wm3, bm3):
    n, h = xs.shape
    d1, d2, d3 = Wm1.shape[1], Wm2.shape[1], Wm3.shape[1]
    grid = n // _B
    return pl.pallas_call(
        _post_body,
        grid=(grid,),
        in_specs=[_parts_spec(h), _rows((_B, h)), _full((1, h)),
                  _full((h, d1)), _full((1, d1)),
                  _full((d1, d2)), _full((1, d2)),
                  _full((d2, d3)), _full((1, d3))],
        out_specs=_rows((_B, d3)),
        out_shape=jax.ShapeDtypeStruct((n, d3), jnp.float32),
    )(parts, xs, b.reshape(1, h), Wm1, bm1.reshape(1, d1),
      Wm2, bm2.reshape(1, d2), Wm3, bm3.reshape(1, d3))


@functools.lru_cache(maxsize=None)
def _make_agg(nchunk, acc_rows, h):
    rows_per_tile = acc_rows // _NS
    mesh = plsc.VectorSubcoreMesh(core_axis_name="c", subcore_axis_name="s")

    @functools.partial(
        pl.kernel,
        mesh=mesh,
        out_type=jax.ShapeDtypeStruct((_NC, acc_rows, h), jnp.float32),
        scratch_types=[
            pltpu.VMEM((_CH, h), jnp.float32),
            pltpu.VMEM((_CH, h), jnp.float32),
            pltpu.VMEM((2, _CH), jnp.int32),
            pltpu.VMEM((2, _CH), jnp.int32),
            pltpu.VMEM_SHARED((acc_rows, h), jnp.float32),
            pltpu.SemaphoreType.DMA,
            pltpu.SemaphoreType.DMA,
            pltpu.SemaphoreType.DMA,
            pltpu.SemaphoreType.DMA,
        ],
    )
    def agg(msg_hbm, idxb_hbm, zeros_hbm, out_hbm,
            rows0_v, rows1_v, idx0_v, idx1_v, acc,
            rsem0, rsem1, isem0, isem1):
        c = lax.axis_index("c")
        s = lax.axis_index("s")
        wid = c * _NS + s
        bufs = ((rows0_v, rsem0, idx0_v, isem0),
                (rows1_v, rsem1, idx1_v, isem1))

        def wait_rows(buf, sem):
            pltpu.make_async_copy(msg_hbm.at[pl.ds(0, _CH)], buf, sem).wait()

        def wait_idx(ibuf, isem):
            pltpu.make_async_copy(idxb_hbm.at[0, 0], ibuf, isem).wait()

        pltpu.async_copy(idxb_hbm.at[wid, 0], idx0_v, isem0)
        pltpu.async_copy(idxb_hbm.at[wid, 1], idx1_v, isem1)
        pltpu.sync_copy(zeros_hbm, acc.at[pl.ds(s * rows_per_tile, rows_per_tile)])
        wait_idx(idx0_v, isem0)
        pltpu.async_copy(msg_hbm.at[idx0_v.at[0]], rows0_v, rsem0)
        plsc.subcore_barrier()

        def slot(j, buf, rsem, ibuf, isem, nbuf, nrsem, nibuf, nisem):
            @pl.when(j + 1 < nchunk)
            def _():
                wait_idx(nibuf, nisem)
                pltpu.async_copy(msg_hbm.at[nibuf.at[0]], nbuf, nrsem)
            wait_rows(buf, rsem)
            pltpu.sync_copy(buf, acc.at[ibuf.at[1]], add=True)

            @pl.when(j + 2 < nchunk)
            def _():
                pltpu.async_copy(idxb_hbm.at[wid, j + 2], ibuf, isem)

        def body(g, carry):
            for b in (0, 1):
                j = 2 * g + b

                @pl.when(j < nchunk)
                def _():
                    slot(j, *bufs[b], *bufs[1 - b])
            return carry

        lax.fori_loop(0, (nchunk + 1) // 2, body, 0)
        plsc.subcore_barrier()
        pltpu.sync_copy(acc.at[pl.ds(s * rows_per_tile, rows_per_tile)],
                        out_hbm.at[c, pl.ds(s * rows_per_tile, rows_per_tile)])

    return agg


def kernel(x, edge_index, Wn1, Ws1, Wimp1, bimp1, b1, Wn2, Ws2, Wimp2, bimp2,
           b2, Wm1, bm1, Wm2, bm2, Wm3, bm3):
    n, d = x.shape
    e = edge_index.shape[1]
    h = Wn1.shape[1]

    nchunk = -(-e // (_NW * _CH))
    ep = _NW * _CH * nchunk
    acc_rows = (_NS * 8) * (-(-(n + 1) // (_NS * 8)))

    src = edge_index[0].astype(jnp.int32)
    dst = edge_index[1].astype(jnp.int32)
    pad = ep - e
    srcb = jnp.concatenate([src, jnp.zeros((pad,), jnp.int32)]).reshape(
        _NW, nchunk, _CH)
    dstb = jnp.concatenate([dst, jnp.full((pad,), n, jnp.int32)]).reshape(
        _NW, nchunk, _CH)
    idxb = jnp.stack([srcb, dstb], axis=2)
    zeros = jnp.zeros((acc_rows // _NS, h), jnp.float32)

    agg_fn = _make_agg(nchunk, acc_rows, h)

    msg1, xs1 = _pre_call(x, Wn1, Ws1, Wimp1, bimp1)
    parts1 = agg_fn(msg1, idxb, zeros)
    msg2, xs2 = _mid_call(parts1, xs1, b1, Wn2, Ws2, Wimp2, bimp2)
    parts2 = agg_fn(msg2, idxb, zeros)
    return _post_call(parts2, xs2, b2, Wm1, bm1, Wm2, bm2, Wm3, bm3)


# speedup vs baseline: 9.2978x; 1.5432x over previous
import functools

import jax
import jax.numpy as jnp
from jax import lax
from jax.experimental import pallas as pl
from jax.experimental.pallas import tpu as pltpu
from jax.experimental.pallas import tpu_sc as plsc

_NC = 2
_NS = 16
_NW = _NC * _NS
_CH = 128

_SPLIT0 = 109.0 / 157.0

_B = 1000


def _sigmoid(v):
    return 1.0 / (1.0 + jnp.exp(-v))


def _pre_body(x_ref, wn_ref, ws_ref, wimp_ref, bimp_ref, msg_ref, xs_ref):
    x = x_ref[...]
    xn = jnp.dot(x, wn_ref[...], preferred_element_type=jnp.float32)
    logit = jnp.dot(xn, wimp_ref[...], preferred_element_type=jnp.float32) + bimp_ref[...]
    msg_ref[...] = _sigmoid(logit) * jnp.maximum(xn, 0.0)
    xs_ref[...] = jnp.dot(x, ws_ref[...], preferred_element_type=jnp.float32)


def _mid_body(parts_ref, xs_ref, b_ref, wn_ref, ws_ref, wimp_ref, bimp_ref,
              msg_ref, xs2_ref):
    p = parts_ref[...]
    h = jnp.maximum(p[0] + p[1] + xs_ref[...] + b_ref[...], 0.0)
    xn = jnp.dot(h, wn_ref[...], preferred_element_type=jnp.float32)
    logit = jnp.dot(xn, wimp_ref[...], preferred_element_type=jnp.float32) + bimp_ref[...]
    msg_ref[...] = _sigmoid(logit) * jnp.maximum(xn, 0.0)
    xs2_ref[...] = jnp.dot(h, ws_ref[...], preferred_element_type=jnp.float32)


def _post_body(parts_ref, xs_ref, b_ref, wm1_ref, bm1_ref, wm2_ref, bm2_ref,
               wm3_ref, bm3_ref, out_ref):
    p = parts_ref[...]
    h = jnp.maximum(p[0] + p[1] + xs_ref[...] + b_ref[...], 0.0)
    m = jnp.maximum(jnp.dot(h, wm1_ref[...], preferred_element_type=jnp.float32)
                    + bm1_ref[...], 0.0)
    m = jnp.maximum(jnp.dot(m, wm2_ref[...], preferred_element_type=jnp.float32)
                    + bm2_ref[...], 0.0)
    out_ref[...] = _sigmoid(
        jnp.dot(m, wm3_ref[...], preferred_element_type=jnp.float32)
        + bm3_ref[...])


def _full(shape):
    return pl.BlockSpec(shape, lambda i: (0,) * len(shape))


def _rows(shape):
    return pl.BlockSpec(shape, lambda i: (i,) + (0,) * (len(shape) - 1))


def _parts_spec(h):
    return pl.BlockSpec((_NC, _B, h), lambda i: (0, i, 0))


def _pre_call(x, Wn, Ws, Wimp, bimp):
    n, d = x.shape
    h = Wn.shape[1]
    grid = n // _B
    return pl.pallas_call(
        _pre_body,
        grid=(grid,),
        in_specs=[_rows((_B, d)), _full((d, h)), _full((d, h)),
                  _full((d, 1)), _full((1, 1))],
        out_specs=[_rows((_B, h)), _rows((_B, h))],
        out_shape=[jax.ShapeDtypeStruct((n, h), jnp.float32),
                   jax.ShapeDtypeStruct((n, h), jnp.float32)],
    )(x, Wn, Ws, Wimp.reshape(d, 1), bimp.reshape(1, 1))


def _mid_call(parts, xs, b, Wn, Ws, Wimp, bimp):
    n, h = xs.shape
    h2 = Wn.shape[1]
    grid = n // _B
    return pl.pallas_call(
        _mid_body,
        grid=(grid,),
        in_specs=[_parts_spec(h), _rows((_B, h)), _full((1, h)),
                  _full((h, h2)), _full((h, h2)), _full((h2, 1)), _full((1, 1))],
        out_specs=[_rows((_B, h2)), _rows((_B, h2))],
        out_shape=[jax.ShapeDtypeStruct((n, h2), jnp.float32),
                   jax.ShapeDtypeStruct((n, h2), jnp.float32)],
    )(parts, xs, b.reshape(1, h), Wn, Ws, Wimp.reshape(h, 1),
      bimp.reshape(1, 1))


def _post_call(parts, xs, b, Wm1, bm1, Wm2, bm2, Wm3, bm3):
    n, h = xs.shape
    d1, d2, d3 = Wm1.shape[1], Wm2.shape[1], Wm3.shape[1]
    grid = n // _B
    return pl.pallas_call(
        _post_body,
        grid=(grid,),
        in_specs=[_parts_spec(h), _rows((_B, h)), _full((1, h)),
                  _full((h, d1)), _full((1, d1)),
                  _full((d1, d2)), _full((1, d2)),
                  _full((d2, d3)), _full((1, d3))],
        out_specs=_rows((_B, d3)),
        out_shape=jax.ShapeDtypeStruct((n, d3), jnp.float32),
    )(parts, xs, b.reshape(1, h), Wm1, bm1.reshape(1, d1),
      Wm2, bm2.reshape(1, d2), Wm3, bm3.reshape(1, d3))


@functools.lru_cache(maxsize=None)
def _make_agg(q0, q1, acc_rows, h):
    rows_per_tile = acc_rows // _NS
    mesh = plsc.VectorSubcoreMesh(core_axis_name="c", subcore_axis_name="s")

    @functools.partial(
        pl.kernel,
        mesh=mesh,
        out_type=jax.ShapeDtypeStruct((_NC, acc_rows, h), jnp.float32),
        scratch_types=[
            pltpu.VMEM((_CH, h), jnp.float32),
            pltpu.VMEM((_CH, h), jnp.float32),
            pltpu.VMEM((2, _CH), jnp.int32),
            pltpu.VMEM((2, _CH), jnp.int32),
            pltpu.VMEM_SHARED((acc_rows, h), jnp.float32),
            pltpu.SemaphoreType.DMA,
            pltpu.SemaphoreType.DMA,
            pltpu.SemaphoreType.DMA,
            pltpu.SemaphoreType.DMA,
        ],
    )
    def agg(msg_hbm, idxb_hbm, zeros_hbm, out_hbm,
            rows0_v, rows1_v, idx0_v, idx1_v, acc,
            rsem0, rsem1, isem0, isem1):
        c = lax.axis_index("c")
        s = lax.axis_index("s")
        base = jnp.where(c == 0, s * q0, _NS * q0 + s * q1)
        my_n = jnp.where(c == 0, q0, q1)
        bufs = ((rows0_v, rsem0, idx0_v, isem0),
                (rows1_v, rsem1, idx1_v, isem1))

        def wait_rows(buf, sem):
            pltpu.make_async_copy(msg_hbm.at[pl.ds(0, _CH)], buf, sem).wait()

        def wait_idx(ibuf, isem):
            pltpu.make_async_copy(idxb_hbm.at[0], ibuf, isem).wait()

        pltpu.async_copy(idxb_hbm.at[base], idx0_v, isem0)

        @pl.when(my_n > 1)
        def _():
            pltpu.async_copy(idxb_hbm.at[base + 1], idx1_v, isem1)
        pltpu.sync_copy(zeros_hbm, acc.at[pl.ds(s * rows_per_tile, rows_per_tile)])
        wait_idx(idx0_v, isem0)
        pltpu.async_copy(msg_hbm.at[idx0_v.at[0]], rows0_v, rsem0)
        plsc.subcore_barrier()

        def slot(j, buf, rsem, ibuf, isem, nbuf, nrsem, nibuf, nisem):
            @pl.when(j + 1 < my_n)
            def _():
                wait_idx(nibuf, nisem)
                pltpu.async_copy(msg_hbm.at[nibuf.at[0]], nbuf, nrsem)
            wait_rows(buf, rsem)
            pltpu.sync_copy(buf, acc.at[ibuf.at[1]], add=True)

            @pl.when(j + 2 < my_n)
            def _():
                pltpu.async_copy(idxb_hbm.at[base + j + 2], ibuf, isem)

        def body(g, carry):
            for b in (0, 1):
                j = 2 * g + b

                @pl.when(j < my_n)
                def _():
                    slot(j, *bufs[b], *bufs[1 - b])
            return carry

        lax.fori_loop(0, (max(q0, q1) + 1) // 2, body, 0)
        plsc.subcore_barrier()
        pltpu.sync_copy(acc.at[pl.ds(s * rows_per_tile, rows_per_tile)],
                        out_hbm.at[c, pl.ds(s * rows_per_tile, rows_per_tile)])

    return agg


def kernel(x, edge_index, Wn1, Ws1, Wimp1, bimp1, b1, Wn2, Ws2, Wimp2, bimp2,
           b2, Wm1, bm1, Wm2, bm2, Wm3, bm3):
    n, d = x.shape
    e = edge_index.shape[1]
    h = Wn1.shape[1]

    nchunks = -(-e // _CH)
    per_pair = -(-nchunks // _NS)
    q0 = max(1, min(per_pair - 1, round(per_pair * _SPLIT0)))
    q1 = per_pair - q0
    tot = per_pair * _NS
    ep = tot * _CH
    acc_rows = (_NS * 8) * (-(-(n + 1) // (_NS * 8)))

    src = edge_index[0].astype(jnp.int32)
    dst = edge_index[1].astype(jnp.int32)
    pad = ep - e
    srcb = jnp.concatenate([src, jnp.zeros((pad,), jnp.int32)]).reshape(
        tot, _CH)
    dstb = jnp.concatenate([dst, jnp.full((pad,), n, jnp.int32)]).reshape(
        tot, _CH)
    idxb = jnp.stack([srcb, dstb], axis=1)
    zeros = jnp.zeros((acc_rows // _NS, h), jnp.float32)

    agg_fn = _make_agg(q0, q1, acc_rows, h)

    msg1, xs1 = _pre_call(x, Wn1, Ws1, Wimp1, bimp1)
    parts1 = agg_fn(msg1, idxb, zeros)
    msg2, xs2 = _mid_call(parts1, xs1, b1, Wn2, Ws2, Wimp2, bimp2)
    parts2 = agg_fn(msg2, idxb, zeros)
    return _post_call(parts2, xs2, b2, Wm1, bm1, Wm2, bm2, Wm3, bm3)


# speedup vs baseline: 9.7221x; 1.0456x over previous
import functools

import jax
import jax.numpy as jnp
from jax import lax
from jax.experimental import pallas as pl
from jax.experimental.pallas import tpu as pltpu
from jax.experimental.pallas import tpu_sc as plsc

_NC = 2
_NS = 16
_NW = _NC * _NS
_CH = 128

_SPLIT0 = 105.0 / 157.0

_B = 1000


def _sigmoid(v):
    return 1.0 / (1.0 + jnp.exp(-v))


def _pre_body(x_ref, wn_ref, ws_ref, wimp_ref, bimp_ref, msg_ref, xs_ref):
    x = x_ref[...]
    xn = jnp.dot(x, wn_ref[...], preferred_element_type=jnp.float32)
    logit = jnp.dot(xn, wimp_ref[...], preferred_element_type=jnp.float32) + bimp_ref[...]
    msg_ref[...] = _sigmoid(logit) * jnp.maximum(xn, 0.0)
    xs_ref[...] = jnp.dot(x, ws_ref[...], preferred_element_type=jnp.float32)


def _mid_body(parts_ref, xs_ref, b_ref, wn_ref, ws_ref, wimp_ref, bimp_ref,
              msg_ref, xs2_ref):
    p = parts_ref[...]
    h = jnp.maximum(p[0] + p[1] + xs_ref[...] + b_ref[...], 0.0)
    xn = jnp.dot(h, wn_ref[...], preferred_element_type=jnp.float32)
    logit = jnp.dot(xn, wimp_ref[...], preferred_element_type=jnp.float32) + bimp_ref[...]
    msg_ref[...] = _sigmoid(logit) * jnp.maximum(xn, 0.0)
    xs2_ref[...] = jnp.dot(h, ws_ref[...], preferred_element_type=jnp.float32)


def _post_body(parts_ref, xs_ref, b_ref, wm1_ref, bm1_ref, wm2_ref, bm2_ref,
               wm3_ref, bm3_ref, out_ref):
    p = parts_ref[...]
    h = jnp.maximum(p[0] + p[1] + xs_ref[...] + b_ref[...], 0.0)
    m = jnp.maximum(jnp.dot(h, wm1_ref[...], preferred_element_type=jnp.float32)
                    + bm1_ref[...], 0.0)
    m = jnp.maximum(jnp.dot(m, wm2_ref[...], preferred_element_type=jnp.float32)
                    + bm2_ref[...], 0.0)
    out_ref[...] = _sigmoid(
        jnp.dot(m, wm3_ref[...], preferred_element_type=jnp.float32)
        + bm3_ref[...])


def _full(shape):
    return pl.BlockSpec(shape, lambda i: (0,) * len(shape))


def _rows(shape):
    return pl.BlockSpec(shape, lambda i: (i,) + (0,) * (len(shape) - 1))


def _parts_spec(h):
    return pl.BlockSpec((_NC, _B, h), lambda i: (0, i, 0))


def _pre_call(x, Wn, Ws, Wimp, bimp):
    n, d = x.shape
    h = Wn.shape[1]
    grid = n // _B
    return pl.pallas_call(
        _pre_body,
        grid=(grid,),
        in_specs=[_rows((_B, d)), _full((d, h)), _full((d, h)),
                  _full((d, 1)), _full((1, 1))],
        out_specs=[_rows((_B, h)), _rows((_B, h))],
        out_shape=[jax.ShapeDtypeStruct((n, h), jnp.float32),
                   jax.ShapeDtypeStruct((n, h), jnp.float32)],
    )(x, Wn, Ws, Wimp.reshape(d, 1), bimp.reshape(1, 1))


def _mid_call(parts, xs, b, Wn, Ws, Wimp, bimp):
    n, h = xs.shape
    h2 = Wn.shape[1]
    grid = n // _B
    return pl.pallas_call(
        _mid_body,
        grid=(grid,),
        in_specs=[_parts_spec(h), _rows((_B, h)), _full((1, h)),
                  _full((h, h2)), _full((h, h2)), _full((h2, 1)), _full((1, 1))],
        out_specs=[_rows((_B, h2)), _rows((_B, h2))],
        out_shape=[jax.ShapeDtypeStruct((n, h2), jnp.float32),
                   jax.ShapeDtypeStruct((n, h2), jnp.float32)],
    )(parts, xs, b.reshape(1, h), Wn, Ws, Wimp.reshape(h, 1),
      bimp.reshape(1, 1))


def _post_call(parts, xs, b, Wm1, bm1, Wm2, bm2, Wm3, bm3):
    n, h = xs.shape
    d1, d2, d3 = Wm1.shape[1], Wm2.shape[1], Wm3.shape[1]
    grid = n // _B
    return pl.pallas_call(
        _post_body,
        grid=(grid,),
        in_specs=[_parts_spec(h), _rows((_B, h)), _full((1, h)),
                  _full((h, d1)), _full((1, d1)),
                  _full((d1, d2)), _full((1, d2)),
                  _full((d2, d3)), _full((1, d3))],
        out_specs=_rows((_B, d3)),
        out_shape=jax.ShapeDtypeStruct((n, d3), jnp.float32),
    )(parts, xs, b.reshape(1, h), Wm1, bm1.reshape(1, d1),
      Wm2, bm2.reshape(1, d2), Wm3, bm3.reshape(1, d3))


@functools.lru_cache(maxsize=None)
def _make_agg(q0, q1, acc_rows, h):
    rows_per_tile = acc_rows // _NS
    mesh = plsc.VectorSubcoreMesh(core_axis_name="c", subcore_axis_name="s")

    @functools.partial(
        pl.kernel,
        mesh=mesh,
        out_type=jax.ShapeDtypeStruct((_NC, acc_rows, h), jnp.float32),
        scratch_types=[
            pltpu.VMEM((_CH, h), jnp.float32),
            pltpu.VMEM((_CH, h), jnp.float32),
            pltpu.VMEM((2, _CH), jnp.int32),
            pltpu.VMEM((2, _CH), jnp.int32),
            pltpu.VMEM_SHARED((acc_rows, h), jnp.float32),
            pltpu.SemaphoreType.DMA,
            pltpu.SemaphoreType.DMA,
            pltpu.SemaphoreType.DMA,
            pltpu.SemaphoreType.DMA,
        ],
    )
    def agg(msg_hbm, srcb_hbm, dstb_hbm, zeros_hbm, out_hbm,
            rows0_v, rows1_v, idx0_v, idx1_v, acc,
            rsem0, rsem1, isem0, isem1):
        c = lax.axis_index("c")
        s = lax.axis_index("s")
        base = jnp.where(c == 0, s * q0, _NS * q0 + s * q1)
        my_n = jnp.where(c == 0, q0, q1)
        bufs = ((rows0_v, rsem0, idx0_v, isem0),
                (rows1_v, rsem1, idx1_v, isem1))

        def wait_rows(buf, sem):
            pltpu.make_async_copy(msg_hbm.at[pl.ds(0, _CH)], buf, sem).wait()

        def fetch_idx(j, ibuf, isem):
            pltpu.async_copy(srcb_hbm.at[j], ibuf.at[0], isem)
            pltpu.async_copy(dstb_hbm.at[j], ibuf.at[1], isem)

        def wait_idx(ibuf, isem):
            pltpu.make_async_copy(srcb_hbm.at[0], ibuf.at[0], isem).wait()
            pltpu.make_async_copy(srcb_hbm.at[0], ibuf.at[1], isem).wait()

        fetch_idx(base, idx0_v, isem0)

        @pl.when(my_n > 1)
        def _():
            fetch_idx(base + 1, idx1_v, isem1)
        pltpu.sync_copy(zeros_hbm, acc.at[pl.ds(s * rows_per_tile, rows_per_tile)])
        wait_idx(idx0_v, isem0)
        pltpu.async_copy(msg_hbm.at[idx0_v.at[0]], rows0_v, rsem0)
        plsc.subcore_barrier()

        def slot(j, buf, rsem, ibuf, isem, nbuf, nrsem, nibuf, nisem):
            @pl.when(j + 1 < my_n)
            def _():
                wait_idx(nibuf, nisem)
                pltpu.async_copy(msg_hbm.at[nibuf.at[0]], nbuf, nrsem)
            wait_rows(buf, rsem)
            pltpu.sync_copy(buf, acc.at[ibuf.at[1]], add=True)

            @pl.when(j + 2 < my_n)
            def _():
                fetch_idx(base + j + 2, ibuf, isem)

        def body(g, carry):
            for b in (0, 1):
                j = 2 * g + b

                @pl.when(j < my_n)
                def _():
                    slot(j, *bufs[b], *bufs[1 - b])
            return carry

        lax.fori_loop(0, (max(q0, q1) + 1) // 2, body, 0)
        plsc.subcore_barrier()
        pltpu.sync_copy(acc.at[pl.ds(s * rows_per_tile, rows_per_tile)],
                        out_hbm.at[c, pl.ds(s * rows_per_tile, rows_per_tile)])

    return agg


def kernel(x, edge_index, Wn1, Ws1, Wimp1, bimp1, b1, Wn2, Ws2, Wimp2, bimp2,
           b2, Wm1, bm1, Wm2, bm2, Wm3, bm3):
    n, d = x.shape
    e = edge_index.shape[1]
    h = Wn1.shape[1]

    nchunks = -(-e // _CH)
    per_pair = -(-nchunks // _NS)
    q0 = max(1, min(per_pair - 1, round(per_pair * _SPLIT0)))
    q1 = per_pair - q0
    tot = per_pair * _NS
    ep = tot * _CH
    acc_rows = (_NS * 8) * (-(-(n + 1) // (_NS * 8)))

    src = edge_index[0].astype(jnp.int32)
    dst = edge_index[1].astype(jnp.int32)
    pad = ep - e
    srcb = jnp.concatenate([src, jnp.zeros((pad,), jnp.int32)]).reshape(
        tot, _CH)
    dstb = jnp.concatenate([dst, jnp.full((pad,), n, jnp.int32)]).reshape(
        tot, _CH)
    zeros = jnp.zeros((acc_rows // _NS, h), jnp.float32)

    agg_fn = _make_agg(q0, q1, acc_rows, h)

    msg1, xs1 = _pre_call(x, Wn1, Ws1, Wimp1, bimp1)
    parts1 = agg_fn(msg1, srcb, dstb, zeros)
    msg2, xs2 = _mid_call(parts1, xs1, b1, Wn2, Ws2, Wimp2, bimp2)
    parts2 = agg_fn(msg2, srcb, dstb, zeros)
    return _post_call(parts2, xs2, b2, Wm1, bm1, Wm2, bm2, Wm3, bm3)


# speedup vs baseline: 10.3928x; 1.0690x over previous
import functools

import jax
import jax.numpy as jnp
from jax import lax
from jax.experimental import pallas as pl
from jax.experimental.pallas import tpu as pltpu
from jax.experimental.pallas import tpu_sc as plsc

_NC = 2
_NS = 16
_NW = _NC * _NS
_CH = 128
_NBUF = 3

_SPLIT0 = 105.0 / 157.0

_B = 1000


def _sigmoid(v):
    return 1.0 / (1.0 + jnp.exp(-v))


def _pre_body(x_ref, wn_ref, ws_ref, wimp_ref, bimp_ref, msg_ref, xs_ref):
    x = x_ref[...]
    xn = jnp.dot(x, wn_ref[...], preferred_element_type=jnp.float32)
    logit = jnp.dot(xn, wimp_ref[...], preferred_element_type=jnp.float32) + bimp_ref[...]
    msg_ref[...] = _sigmoid(logit) * jnp.maximum(xn, 0.0)
    xs_ref[...] = jnp.dot(x, ws_ref[...], preferred_element_type=jnp.float32)


def _mid_body(parts_ref, xs_ref, b_ref, wn_ref, ws_ref, wimp_ref, bimp_ref,
              msg_ref, xs2_ref):
    p = parts_ref[...]
    h = jnp.maximum(p[0] + p[1] + xs_ref[...] + b_ref[...], 0.0)
    xn = jnp.dot(h, wn_ref[...], preferred_element_type=jnp.float32)
    logit = jnp.dot(xn, wimp_ref[...], preferred_element_type=jnp.float32) + bimp_ref[...]
    msg_ref[...] = _sigmoid(logit) * jnp.maximum(xn, 0.0)
    xs2_ref[...] = jnp.dot(h, ws_ref[...], preferred_element_type=jnp.float32)


def _post_body(parts_ref, xs_ref, b_ref, wm1_ref, bm1_ref, wm2_ref, bm2_ref,
               wm3_ref, bm3_ref, out_ref):
    p = parts_ref[...]
    h = jnp.maximum(p[0] + p[1] + xs_ref[...] + b_ref[...], 0.0)
    m = jnp.maximum(jnp.dot(h, wm1_ref[...], preferred_element_type=jnp.float32)
                    + bm1_ref[...], 0.0)
    m = jnp.maximum(jnp.dot(m, wm2_ref[...], preferred_element_type=jnp.float32)
                    + bm2_ref[...], 0.0)
    out_ref[...] = _sigmoid(
        jnp.dot(m, wm3_ref[...], preferred_element_type=jnp.float32)
        + bm3_ref[...])


def _full(shape):
    return pl.BlockSpec(shape, lambda i: (0,) * len(shape))


def _rows(shape):
    return pl.BlockSpec(shape, lambda i: (i,) + (0,) * (len(shape) - 1))


def _parts_spec(h):
    return pl.BlockSpec((_NC, _B, h), lambda i: (0, i, 0))


def _pre_call(x, Wn, Ws, Wimp, bimp):
    n, d = x.shape
    h = Wn.shape[1]
    grid = n // _B
    return pl.pallas_call(
        _pre_body,
        grid=(grid,),
        in_specs=[_rows((_B, d)), _full((d, h)), _full((d, h)),
                  _full((d, 1)), _full((1, 1))],
        out_specs=[_rows((_B, h)), _rows((_B, h))],
        out_shape=[jax.ShapeDtypeStruct((n, h), jnp.float32),
                   jax.ShapeDtypeStruct((n, h), jnp.float32)],
    )(x, Wn, Ws, Wimp.reshape(d, 1), bimp.reshape(1, 1))


def _mid_call(parts, xs, b, Wn, Ws, Wimp, bimp):
    n, h = xs.shape
    h2 = Wn.shape[1]
    grid = n // _B
    return pl.pallas_call(
        _mid_body,
        grid=(grid,),
        in_specs=[_parts_spec(h), _rows((_B, h)), _full((1, h)),
                  _full((h, h2)), _full((h, h2)), _full((h2, 1)), _full((1, 1))],
        out_specs=[_rows((_B, h2)), _rows((_B, h2))],
        out_shape=[jax.ShapeDtypeStruct((n, h2), jnp.float32),
                   jax.ShapeDtypeStruct((n, h2), jnp.float32)],
    )(parts, xs, b.reshape(1, h), Wn, Ws, Wimp.reshape(h, 1),
      bimp.reshape(1, 1))


def _post_call(parts, xs, b, Wm1, bm1, Wm2, bm2, Wm3, bm3):
    n, h = xs.shape
    d1, d2, d3 = Wm1.shape[1], Wm2.shape[1], Wm3.shape[1]
    grid = n // _B
    return pl.pallas_call(
        _post_body,
        grid=(grid,),
        in_specs=[_parts_spec(h), _rows((_B, h)), _full((1, h)),
                  _full((h, d1)), _full((1, d1)),
                  _full((d1, d2)), _full((1, d2)),
                  _full((d2, d3)), _full((1, d3))],
        out_specs=_rows((_B, d3)),
        out_shape=jax.ShapeDtypeStruct((n, d3), jnp.float32),
    )(parts, xs, b.reshape(1, h), Wm1, bm1.reshape(1, d1),
      Wm2, bm2.reshape(1, d2), Wm3, bm3.reshape(1, d3))


@functools.lru_cache(maxsize=None)
def _make_agg(q0, q1, acc_rows, h):
    rows_per_tile = acc_rows // _NS
    mesh = plsc.VectorSubcoreMesh(core_axis_name="c", subcore_axis_name="s")

    @functools.partial(
        pl.kernel,
        mesh=mesh,
        out_type=jax.ShapeDtypeStruct((_NC, acc_rows, h), jnp.float32),
        scratch_types=(
            [pltpu.VMEM((_CH, h), jnp.float32) for _ in range(_NBUF)]
            + [pltpu.VMEM((2, _CH), jnp.int32) for _ in range(_NBUF)]
            + [pltpu.VMEM_SHARED((acc_rows, h), jnp.float32)]
            + [pltpu.SemaphoreType.DMA for _ in range(2 * _NBUF)]
        ),
    )
    def agg(msg_hbm, srcb_hbm, dstb_hbm, zeros_hbm, out_hbm, *scr):
        rows = scr[:_NBUF]
        ibufs = scr[_NBUF:2 * _NBUF]
        acc = scr[2 * _NBUF]
        rsems = scr[2 * _NBUF + 1:3 * _NBUF + 1]
        isems = scr[3 * _NBUF + 1:]
        c = lax.axis_index("c")
        s = lax.axis_index("s")
        base = jnp.where(c == 0, s * q0, _NS * q0 + s * q1)
        my_n = jnp.where(c == 0, q0, q1)

        def wait_rows(b):
            pltpu.make_async_copy(msg_hbm.at[pl.ds(0, _CH)], rows[b],
                                  rsems[b]).wait()

        def fetch_idx(j, b):
            pltpu.async_copy(srcb_hbm.at[j], ibufs[b].at[0], isems[b])
            pltpu.async_copy(dstb_hbm.at[j], ibufs[b].at[1], isems[b])

        def gather(b):
            pltpu.make_async_copy(srcb_hbm.at[0], ibufs[b].at[0], isems[b]).wait()
            pltpu.make_async_copy(srcb_hbm.at[0], ibufs[b].at[1], isems[b]).wait()
            pltpu.async_copy(msg_hbm.at[ibufs[b].at[0]], rows[b], rsems[b])

        for k in range(_NBUF):
            @pl.when(k < my_n)
            def _(k=k):
                fetch_idx(base + k, k)
        pltpu.sync_copy(zeros_hbm, acc.at[pl.ds(s * rows_per_tile, rows_per_tile)])
        for k in range(_NBUF - 1):
            @pl.when(k < my_n)
            def _(k=k):
                gather(k)
        plsc.subcore_barrier()

        def slot(j, b):
            @pl.when(j + _NBUF - 1 < my_n)
            def _():
                gather((b + _NBUF - 1) % _NBUF)
            wait_rows(b)
            pltpu.sync_copy(rows[b], acc.at[ibufs[b].at[1]], add=True)

            @pl.when(j + _NBUF < my_n)
            def _():
                fetch_idx(base + j + _NBUF, b)

        def body(g, carry):
            for b in range(_NBUF):
                j = _NBUF * g + b

                @pl.when(j < my_n)
                def _():
                    slot(j, b)
            return carry

        lax.fori_loop(0, -(-max(q0, q1) // _NBUF), body, 0)
        plsc.subcore_barrier()
        pltpu.sync_copy(acc.at[pl.ds(s * rows_per_tile, rows_per_tile)],
                        out_hbm.at[c, pl.ds(s * rows_per_tile, rows_per_tile)])

    return agg


def kernel(x, edge_index, Wn1, Ws1, Wimp1, bimp1, b1, Wn2, Ws2, Wimp2, bimp2,
           b2, Wm1, bm1, Wm2, bm2, Wm3, bm3):
    n, d = x.shape
    e = edge_index.shape[1]
    h = Wn1.shape[1]

    nchunks = -(-e // _CH)
    per_pair = -(-nchunks // _NS)
    q0 = max(1, min(per_pair - 1, round(per_pair * _SPLIT0)))
    q1 = per_pair - q0
    tot = per_pair * _NS
    ep = tot * _CH
    acc_rows = (_NS * 8) * (-(-(n + 1) // (_NS * 8)))

    src = edge_index[0].astype(jnp.int32)
    dst = edge_index[1].astype(jnp.int32)
    pad = ep - e
    srcb = jnp.concatenate([src, jnp.zeros((pad,), jnp.int32)]).reshape(
        tot, _CH)
    dstb = jnp.concatenate([dst, jnp.full((pad,), n, jnp.int32)]).reshape(
        tot, _CH)
    zeros = jnp.zeros((acc_rows // _NS, h), jnp.float32)

    agg_fn = _make_agg(q0, q1, acc_rows, h)

    msg1, xs1 = _pre_call(x, Wn1, Ws1, Wimp1, bimp1)
    parts1 = agg_fn(msg1, srcb, dstb, zeros)
    msg2, xs2 = _mid_call(parts1, xs1, b1, Wn2, Ws2, Wimp2, bimp2)
    parts2 = agg_fn(msg2, srcb, dstb, zeros)
    return _post_call(parts2, xs2, b2, Wm1, bm1, Wm2, bm2, Wm3, bm3)


# speedup vs baseline: 12.0146x; 1.1561x over previous
import functools

import jax
import jax.numpy as jnp
from jax import lax
from jax.experimental import pallas as pl
from jax.experimental.pallas import tpu as pltpu
from jax.experimental.pallas import tpu_sc as plsc

_NC = 2
_NS = 16
_NW = _NC * _NS
_CH = 120
_NBUF = 3

_SPLIT0 = 105.0 / 157.0

_B = 1000


def _sigmoid(v):
    return 1.0 / (1.0 + jnp.exp(-v))


def _pre_body(x_ref, wn_ref, ws_ref, wimp_ref, bimp_ref, msg_ref, xs_ref):
    x = x_ref[...]
    xn = jnp.dot(x, wn_ref[...], preferred_element_type=jnp.float32)
    logit = jnp.dot(xn, wimp_ref[...], preferred_element_type=jnp.float32) + bimp_ref[...]
    msg_ref[...] = _sigmoid(logit) * jnp.maximum(xn, 0.0)
    xs_ref[...] = jnp.dot(x, ws_ref[...], preferred_element_type=jnp.float32)


def _mid_body(parts_ref, xs_ref, b_ref, wn_ref, ws_ref, wimp_ref, bimp_ref,
              msg_ref, xs2_ref):
    p = parts_ref[...]
    h = jnp.maximum(p[0] + p[1] + xs_ref[...] + b_ref[...], 0.0)
    xn = jnp.dot(h, wn_ref[...], preferred_element_type=jnp.float32)
    logit = jnp.dot(xn, wimp_ref[...], preferred_element_type=jnp.float32) + bimp_ref[...]
    msg_ref[...] = _sigmoid(logit) * jnp.maximum(xn, 0.0)
    xs2_ref[...] = jnp.dot(h, ws_ref[...], preferred_element_type=jnp.float32)


def _post_body(parts_ref, xs_ref, b_ref, wm1_ref, bm1_ref, wm2_ref, bm2_ref,
               wm3_ref, bm3_ref, out_ref):
    p = parts_ref[...]
    h = jnp.maximum(p[0] + p[1] + xs_ref[...] + b_ref[...], 0.0)
    m = jnp.maximum(jnp.dot(h, wm1_ref[...], preferred_element_type=jnp.float32)
                    + bm1_ref[...], 0.0)
    m = jnp.maximum(jnp.dot(m, wm2_ref[...], preferred_element_type=jnp.float32)
                    + bm2_ref[...], 0.0)
    out_ref[...] = _sigmoid(
        jnp.dot(m, wm3_ref[...], preferred_element_type=jnp.float32)
        + bm3_ref[...])


def _full(shape):
    return pl.BlockSpec(shape, lambda i: (0,) * len(shape))


def _rows(shape):
    return pl.BlockSpec(shape, lambda i: (i,) + (0,) * (len(shape) - 1))


def _parts_spec(h):
    return pl.BlockSpec((_NC, _B, h), lambda i: (0, i, 0))


def _pre_call(x, Wn, Ws, Wimp, bimp):
    n, d = x.shape
    h = Wn.shape[1]
    grid = n // _B
    return pl.pallas_call(
        _pre_body,
        grid=(grid,),
        in_specs=[_rows((_B, d)), _full((d, h)), _full((d, h)),
                  _full((d, 1)), _full((1, 1))],
        out_specs=[_rows((_B, h)), _rows((_B, h))],
        out_shape=[jax.ShapeDtypeStruct((n, h), jnp.float32),
                   jax.ShapeDtypeStruct((n, h), jnp.float32)],
    )(x, Wn, Ws, Wimp.reshape(d, 1), bimp.reshape(1, 1))


def _mid_call(parts, xs, b, Wn, Ws, Wimp, bimp):
    n, h = xs.shape
    h2 = Wn.shape[1]
    grid = n // _B
    return pl.pallas_call(
        _mid_body,
        grid=(grid,),
        in_specs=[_parts_spec(h), _rows((_B, h)), _full((1, h)),
                  _full((h, h2)), _full((h, h2)), _full((h2, 1)), _full((1, 1))],
        out_specs=[_rows((_B, h2)), _rows((_B, h2))],
        out_shape=[jax.ShapeDtypeStruct((n, h2), jnp.float32),
                   jax.ShapeDtypeStruct((n, h2), jnp.float32)],
    )(parts, xs, b.reshape(1, h), Wn, Ws, Wimp.reshape(h, 1),
      bimp.reshape(1, 1))


def _post_call(parts, xs, b, Wm1, bm1, Wm2, bm2, Wm3, bm3):
    n, h = xs.shape
    d1, d2, d3 = Wm1.shape[1], Wm2.shape[1], Wm3.shape[1]
    grid = n // _B
    return pl.pallas_call(
        _post_body,
        grid=(grid,),
        in_specs=[_parts_spec(h), _rows((_B, h)), _full((1, h)),
                  _full((h, d1)), _full((1, d1)),
                  _full((d1, d2)), _full((1, d2)),
                  _full((d2, d3)), _full((1, d3))],
        out_specs=_rows((_B, d3)),
        out_shape=jax.ShapeDtypeStruct((n, d3), jnp.float32),
    )(parts, xs, b.reshape(1, h), Wm1, bm1.reshape(1, d1),
      Wm2, bm2.reshape(1, d2), Wm3, bm3.reshape(1, d3))


@functools.lru_cache(maxsize=None)
def _make_agg(q0, q1, acc_rows, h):
    rows_per_tile = acc_rows // _NS
    mesh = plsc.VectorSubcoreMesh(core_axis_name="c", subcore_axis_name="s")

    @functools.partial(
        pl.kernel,
        mesh=mesh,
        out_type=jax.ShapeDtypeStruct((_NC, acc_rows, h), jnp.float32),
        scratch_types=(
            [pltpu.VMEM((_CH, h), jnp.float32) for _ in range(_NBUF)]
            + [pltpu.VMEM((2, _CH), jnp.int32) for _ in range(2 * _NBUF)]
            + [pltpu.VMEM_SHARED((acc_rows, h), jnp.float32)]
            + [pltpu.SemaphoreType.DMA for _ in range(4 * _NBUF)]
        ),
    )
    def agg(msg_hbm, srcb_hbm, dstb_hbm, zeros_hbm, out_hbm, *scr):
        rows = scr[:_NBUF]
        ibufs = scr[_NBUF:3 * _NBUF]
        acc = scr[3 * _NBUF]
        rsems = scr[3 * _NBUF + 1:4 * _NBUF + 1]
        ssems = scr[4 * _NBUF + 1:5 * _NBUF + 1]
        isems = scr[5 * _NBUF + 1:]
        c = lax.axis_index("c")
        s = lax.axis_index("s")
        base = jnp.where(c == 0, s * q0, _NS * q0 + s * q1)
        my_n = jnp.where(c == 0, q0, q1)

        def wait_rows(b):
            pltpu.make_async_copy(msg_hbm.at[pl.ds(0, _CH)], rows[b],
                                  rsems[b]).wait()

        def wait_scatter(b):
            pltpu.make_async_copy(msg_hbm.at[pl.ds(0, _CH)], rows[b],
                                  ssems[b]).wait()

        def fetch_idx(j, ib):
            pltpu.async_copy(srcb_hbm.at[j], ibufs[ib].at[0], isems[ib])
            pltpu.async_copy(dstb_hbm.at[j], ibufs[ib].at[1], isems[ib])

        def gather(b, ib):
            pltpu.make_async_copy(srcb_hbm.at[0], ibufs[ib].at[0],
                                  isems[ib]).wait()
            pltpu.make_async_copy(srcb_hbm.at[0], ibufs[ib].at[1],
                                  isems[ib]).wait()
            pltpu.async_copy(msg_hbm.at[ibufs[ib].at[0]], rows[b], rsems[b])

        for k in range(_NBUF):
            @pl.when(k < my_n)
            def _(k=k):
                fetch_idx(base + k, k)
        pltpu.sync_copy(zeros_hbm, acc.at[pl.ds(s * rows_per_tile, rows_per_tile)])
        for k in range(_NBUF - 1):
            @pl.when(k < my_n)
            def _(k=k):
                gather(k, k)
        plsc.subcore_barrier()

        def slot(j, b, ib):
            pb = (b + _NBUF - 1) % _NBUF

            @pl.when((j >= 1) & (j - 1 < my_n))
            def _():
                wait_scatter(pb)

            @pl.when(j + _NBUF - 1 < my_n)
            def _():
                gather(pb, (ib + _NBUF - 1) % (2 * _NBUF))

            @pl.when(j < my_n)
            def _():
                wait_rows(b)
                pltpu.async_copy(rows[b], acc.at[ibufs[ib].at[1]], ssems[b],
                                 add=True)

            @pl.when(j + _NBUF < my_n)
            def _():
                fetch_idx(base + j + _NBUF, (ib + _NBUF) % (2 * _NBUF))

        def body(g, carry):
            for u in range(2 * _NBUF):
                j = 2 * _NBUF * g + u

                @pl.when(j < my_n + 1)
                def _():
                    slot(j, u % _NBUF, u)
            return carry

        lax.fori_loop(0, -(-(max(q0, q1) + 1) // (2 * _NBUF)), body, 0)
        plsc.subcore_barrier()
        pltpu.sync_copy(acc.at[pl.ds(s * rows_per_tile, rows_per_tile)],
                        out_hbm.at[c, pl.ds(s * rows_per_tile, rows_per_tile)])

    return agg


def kernel(x, edge_index, Wn1, Ws1, Wimp1, bimp1, b1, Wn2, Ws2, Wimp2, bimp2,
           b2, Wm1, bm1, Wm2, bm2, Wm3, bm3):
    n, d = x.shape
    e = edge_index.shape[1]
    h = Wn1.shape[1]

    nchunks = -(-e // _CH)
    per_pair = -(-nchunks // _NS)
    q0 = max(1, min(per_pair - 1, round(per_pair * _SPLIT0)))
    q1 = per_pair - q0
    tot = per_pair * _NS
    ep = tot * _CH
    acc_rows = (_NS * 8) * (-(-(n + 1) // (_NS * 8)))

    src = edge_index[0].astype(jnp.int32)
    dst = edge_index[1].astype(jnp.int32)
    pad = ep - e
    srcb = jnp.concatenate([src, jnp.zeros((pad,), jnp.int32)]).reshape(
        tot, _CH)
    dstb = jnp.concatenate([dst, jnp.full((pad,), n, jnp.int32)]).reshape(
        tot, _CH)
    zeros = jnp.zeros((acc_rows // _NS, h), jnp.float32)

    agg_fn = _make_agg(q0, q1, acc_rows, h)

    msg1, xs1 = _pre_call(x, Wn1, Ws1, Wimp1, bimp1)
    parts1 = agg_fn(msg1, srcb, dstb, zeros)
    msg2, xs2 = _mid_call(parts1, xs1, b1, Wn2, Ws2, Wimp2, bimp2)
    parts2 = agg_fn(msg2, srcb, dstb, zeros)
    return _post_call(parts2, xs2, b2, Wm1, bm1, Wm2, bm2, Wm3, bm3)
